# baseline (speedup 1.0000x reference)
import functools

import jax
import jax.numpy as jnp
from jax import lax
from jax.experimental import pallas as pl
from jax.experimental.pallas import tpu as pltpu
from jax.experimental.pallas import tpu_sc as plsc

_B = 425984
_D = 32
_T = 1000000
_Q = 250000
_NW = 32
_P1_MAIN = 122
_P2_WIN = 104
_BPW = _B // _NW

_mesh = plsc.VectorSubcoreMesh(core_axis_name="core", subcore_axis_name="subcore")
_cp = pltpu.CompilerParams(use_tc_tiling_on_sc=True, needs_layout_passes=False)


def _lane_quads():
    io = lax.iota(jnp.int32, 16)
    return io >> 2, io & 3


def _transpose_window(instage, outstage, ncols):
    io = lax.iota(jnp.int32, 16)
    q_off = io >> 2
    sub = (io & 3) * 32

    def d_step(t, _):
        for u in range(2):
            d = 2 * t + u
            d_splat = jnp.broadcast_to(d, (16,)).astype(jnp.int32)
            col_vec = sub + ((d_splat + io) & 31)
            for r0 in range(0, ncols, 16):
                vals = plsc.load_gather(instage, [d_splat, r0 + io])
                plsc.store_scatter(outstage, [r0 // 4 + q_off, col_vec], vals)
        return _

    lax.fori_loop(0, _D // 2, d_step, None)


def _fused(tableT, idx):
    @pl.kernel(
        out_type=(
            jax.ShapeDtypeStruct((_D, _B), jnp.float32),
            jax.ShapeDtypeStruct((_Q, 128), jnp.float32),
        ),
        mesh=_mesh,
        compiler_params=_cp,
        scratch_types=[
            pltpu.VMEM((_D, 256), jnp.float32),
            pltpu.VMEM((_D, 256), jnp.float32),
            pltpu.VMEM((_D, 256), jnp.float32),
            pltpu.VMEM((64, 128), jnp.float32),
            pltpu.VMEM((64, 128), jnp.float32),
            pltpu.VMEM((_BPW,), jnp.int32),
            pltpu.VMEM((_BPW,), jnp.int32),
            pltpu.VMEM((_BPW,), jnp.int32),
            pltpu.VMEM((128, 128), jnp.float32),
            pltpu.VMEM((128, 128), jnp.float32),
            pltpu.VMEM((_D, 128), jnp.float32),
            pltpu.VMEM((_D, 128), jnp.float32),
            pltpu.VMEM((_D, 64), jnp.float32),
            pltpu.SemaphoreType.DMA,
            pltpu.SemaphoreType.DMA,
            pltpu.SemaphoreType.DMA,
            pltpu.SemaphoreType.DMA,
            pltpu.SemaphoreType.DMA,
            pltpu.SemaphoreType.DMA,
            pltpu.SemaphoreType.DMA,
            pltpu.SemaphoreType.DMA,
            pltpu.SemaphoreType.DMA,
            pltpu.SemaphoreType.REGULAR,
        ],
    )
    def body(t_hbm, i_hbm, o_hbm, t4_hbm,
             in0, in1, in2, tr0, tr1, idx_all, prem_all, rot_all,
             gath0, gath1, ost0, ost1, tailst,
             sem_in0, sem_in1, sem_in2, sem_out0, sem_out1,
             sem_g0, sem_g1, sem_o0, sem_o1, bsem):
        wid = lax.axis_index("subcore") * 2 + lax.axis_index("core")
        ins = (in0, in1, in2)
        trs = (tr0, tr1)
        sem_ins = (sem_in0, sem_in1, sem_in2)
        sem_outs = (sem_out0, sem_out1)

        def in_copy(j, b):
            c = wid + _NW * j
            return pltpu.make_async_copy(
                t_hbm.at[:, pl.ds(256 * c, 256)], ins[b], sem_ins[b])

        def out_copy(j, b):
            c = wid + _NW * j
            return pltpu.make_async_copy(
                trs[b], t4_hbm.at[pl.ds(64 * c, 64), :], sem_outs[b])

        in_copy(0, 0).start()
        in_copy(1, 1).start()
        in_copy(2, 2).start()

        def p1_step(k, _):
            for u in range(6):
                j = 6 * k + u
                b3 = u % 3
                b2 = u % 2
                in_copy(j, b3).wait()

                @pl.when(j >= 2)
                def _():
                    out_copy(j - 2, b2).wait()

                _transpose_window(ins[b3], trs[b2], 256)
                out_copy(j, b2).start()

                @pl.when(j + 3 < _P1_MAIN)
                def _():
                    in_copy(j + 3, b3).start()
            return _

        lax.fori_loop(0, 120 // 6, p1_step, None)
        for j in (120, 121):
            in_copy(j, j % 3).wait()
            out_copy(j - 2, j % 2).wait()
            _transpose_window(ins[j % 3], trs[j % 2], 256)
            out_copy(j, j % 2).start()
        out_copy(120, 0).wait()
        out_copy(121, 1).wait()

        @pl.when(wid < 2)
        def _():
            c = 3904 + wid
            pltpu.sync_copy(t_hbm.at[:, pl.ds(256 * c, 256)], in0)
            _transpose_window(in0, tr0, 256)
            pltpu.sync_copy(tr0, t4_hbm.at[pl.ds(64 * c, 64), :])

        @pl.when(wid == 4)
        def _():
            pltpu.sync_copy(t_hbm.at[:, pl.ds(999936, 64)], tailst)
            _transpose_window(tailst, tr0, 64)
            pltpu.sync_copy(
                tr0.at[pl.ds(0, 16), :], t4_hbm.at[pl.ds(249984, 16), :])

        plsc.subcore_barrier()
        pltpu.core_barrier(bsem, core_axis_name="core")
        plsc.subcore_barrier()

        pltpu.sync_copy(i_hbm.at[pl.ds(wid * _BPW, _BPW)], idx_all)

        def idx_step(g, _):
            v = idx_all[pl.ds(16 * g, 16)]
            prem_all[pl.ds(16 * g, 16)] = (v & 3) * 32
            rot_all[pl.ds(16 * g, 16)] = v & 15
            idx_all[pl.ds(16 * g, 16)] = v >> 2
            return _

        lax.fori_loop(0, _BPW // 16, idx_step, None)

        gaths = (gath0, gath1)
        osts = (ost0, ost1)
        sem_gs = (sem_g0, sem_g1)
        sem_os = (sem_o0, sem_o1)

        def g_copy(j, b):
            return pltpu.make_async_copy(
                t4_hbm.at[idx_all.at[pl.ds(128 * j, 128)]], gaths[b], sem_gs[b])

        def o_copy(j, b):
            col = 128 * (wid * _P2_WIN + j)
            return pltpu.make_async_copy(
                osts[b], o_hbm.at[:, pl.ds(col, 128)], sem_os[b])

        g_copy(0, 0).start()
        g_copy(1, 1).start()

        dd, rr = _lane_quads()

        def p2_step(k, _):
            for b in range(2):
                j = 2 * k + b
                g_copy(j, b).wait()

                @pl.when(j >= 2)
                def _():
                    o_copy(j - 2, b).wait()

                base = 128 * j

                def i_step(t, _, b=b):
                    i0 = 4 * t
                    prem_rep = plsc.load_gather(prem_all, [base + i0 + dd])
                    rot_rep = plsc.load_gather(rot_all, [base + i0 + dd])
                    row_vec = i0 + dd
                    col_vec = jnp.broadcast_to(i0, (16,)).astype(jnp.int32) + dd
                    for d0 in range(0, _D, 4):
                        skew = prem_rep + ((d0 + rr + rot_rep) & 31)
                        vals = plsc.load_gather(gaths[b], [row_vec, skew])
                        plsc.store_scatter(osts[b], [d0 + rr, col_vec], vals)
                    return _

                lax.fori_loop(0, 32, i_step, None)
                o_copy(j, b).start()

                @pl.when(j + 2 < _P2_WIN)
                def _():
                    g_copy(j + 2, b).start()
            return _

        lax.fori_loop(0, _P2_WIN // 2, p2_step, None)
        o_copy(_P2_WIN - 2, 0).wait()
        o_copy(_P2_WIN - 1, 1).wait()

    return body(tableT, idx)


def kernel(variable_embeddings, candidate_indices):
    idx = candidate_indices.astype(jnp.int32)
    outT, _ = _fused(variable_embeddings.T, idx)
    return outT.T


# speedup vs baseline: 1.0205x; 1.0205x over previous
import functools

import jax
import jax.numpy as jnp
from jax import lax
from jax.experimental import pallas as pl
from jax.experimental.pallas import tpu as pltpu
from jax.experimental.pallas import tpu_sc as plsc

_B = 425984
_D = 32
_T = 1000000
_Q = 250000
_NW = 32
_P1_MAIN = 122
_P2_WIN = 104
_BPW = _B // _NW

_mesh = plsc.VectorSubcoreMesh(core_axis_name="core", subcore_axis_name="subcore")
_cp = pltpu.CompilerParams(use_tc_tiling_on_sc=True, needs_layout_passes=False)


def _lane_quads():
    io = lax.iota(jnp.int32, 16)
    return io >> 2, io & 3


def _transpose_window(instage, outstage, ncols):
    io = lax.iota(jnp.int32, 16)
    q_off = io >> 2
    sub = (io & 3) * 32

    def d_step(t, _):
        for u in range(2):
            d = 2 * t + u
            d_splat = jnp.broadcast_to(d, (16,)).astype(jnp.int32)
            col_vec = sub + ((d_splat + io) & 31)
            for r0 in range(0, ncols, 16):
                vals = plsc.load_gather(instage, [d_splat, r0 + io])
                plsc.store_scatter(outstage, [r0 // 4 + q_off, col_vec], vals)
        return _

    lax.fori_loop(0, _D // 2, d_step, None)


def _fused(tableT, idx):
    @pl.kernel(
        out_type=(
            jax.ShapeDtypeStruct((_D, _B), jnp.float32),
            jax.ShapeDtypeStruct((_Q, 128), jnp.float32),
        ),
        mesh=_mesh,
        compiler_params=_cp,
        scratch_types=[
            pltpu.VMEM((_D, 256), jnp.float32),
            pltpu.VMEM((_D, 256), jnp.float32),
            pltpu.VMEM((64, 128), jnp.float32),
            pltpu.VMEM((64, 128), jnp.float32),
            pltpu.VMEM((_BPW,), jnp.int32),
            pltpu.VMEM((_BPW,), jnp.int32),
            pltpu.VMEM((_BPW,), jnp.int32),
            pltpu.VMEM((128, 128), jnp.float32),
            pltpu.VMEM((128, 128), jnp.float32),
            pltpu.VMEM((_D, 128), jnp.float32),
            pltpu.VMEM((_D, 128), jnp.float32),
            pltpu.VMEM((_D, 64), jnp.float32),
            pltpu.SemaphoreType.DMA,
            pltpu.SemaphoreType.DMA,
            pltpu.SemaphoreType.DMA,
            pltpu.SemaphoreType.DMA,
            pltpu.SemaphoreType.DMA,
            pltpu.SemaphoreType.DMA,
            pltpu.SemaphoreType.DMA,
            pltpu.SemaphoreType.DMA,
            pltpu.SemaphoreType.REGULAR,
        ],
    )
    def body(t_hbm, i_hbm, o_hbm, t4_hbm,
             in0, in1, tr0, tr1, idx_all, prem_all, rot_all,
             gath0, gath1, ost0, ost1, tailst,
             sem_in0, sem_in1, sem_out0, sem_out1,
             sem_g0, sem_g1, sem_o0, sem_o1, bsem):
        wid = lax.axis_index("subcore") * 2 + lax.axis_index("core")
        ins = (in0, in1)
        trs = (tr0, tr1)
        sem_ins = (sem_in0, sem_in1)
        sem_outs = (sem_out0, sem_out1)

        def in_copy(j, b):
            c = wid + _NW * j
            return pltpu.make_async_copy(
                t_hbm.at[:, pl.ds(256 * c, 256)], ins[b], sem_ins[b])

        def out_copy(j, b):
            c = wid + _NW * j
            return pltpu.make_async_copy(
                trs[b], t4_hbm.at[pl.ds(64 * c, 64), :], sem_outs[b])

        in_copy(0, 0).start()
        in_copy(1, 1).start()

        def p1_step(k, _):
            for b in range(2):
                j = 2 * k + b
                in_copy(j, b).wait()

                @pl.when(j >= 2)
                def _():
                    out_copy(j - 2, b).wait()

                _transpose_window(ins[b], trs[b], 256)
                out_copy(j, b).start()

                @pl.when(j + 2 < _P1_MAIN)
                def _():
                    in_copy(j + 2, b).start()
            return _

        lax.fori_loop(0, _P1_MAIN // 2, p1_step, None)
        out_copy(_P1_MAIN - 2, 0).wait()
        out_copy(_P1_MAIN - 1, 1).wait()

        @pl.when(wid < 2)
        def _():
            c = 3904 + wid
            pltpu.sync_copy(t_hbm.at[:, pl.ds(256 * c, 256)], in0)
            _transpose_window(in0, tr0, 256)
            pltpu.sync_copy(tr0, t4_hbm.at[pl.ds(64 * c, 64), :])

        @pl.when(wid == 4)
        def _():
            pltpu.sync_copy(t_hbm.at[:, pl.ds(999936, 64)], tailst)
            _transpose_window(tailst, tr0, 64)
            pltpu.sync_copy(
                tr0.at[pl.ds(0, 16), :], t4_hbm.at[pl.ds(249984, 16), :])

        plsc.subcore_barrier()
        pltpu.core_barrier(bsem, core_axis_name="core")
        plsc.subcore_barrier()

        pltpu.sync_copy(i_hbm.at[pl.ds(wid * _BPW, _BPW)], idx_all)

        def idx_step(g, _):
            v = idx_all[pl.ds(16 * g, 16)]
            prem_all[pl.ds(16 * g, 16)] = (v & 3) * 32
            rot_all[pl.ds(16 * g, 16)] = v & 15
            idx_all[pl.ds(16 * g, 16)] = v >> 2
            return _

        lax.fori_loop(0, _BPW // 16, idx_step, None)

        gaths = (gath0, gath1)
        osts = (ost0, ost1)
        sem_gs = (sem_g0, sem_g1)
        sem_os = (sem_o0, sem_o1)

        def g_copy(j, b):
            return pltpu.make_async_copy(
                t4_hbm.at[idx_all.at[pl.ds(128 * j, 128)]], gaths[b], sem_gs[b])

        def o_copy(j, b):
            col = 128 * (wid * _P2_WIN + j)
            return pltpu.make_async_copy(
                osts[b], o_hbm.at[:, pl.ds(col, 128)], sem_os[b])

        g_copy(0, 0).start()
        g_copy(1, 1).start()

        dd, rr = _lane_quads()

        def p2_step(k, _):
            for b in range(2):
                j = 2 * k + b
                g_copy(j, b).wait()

                @pl.when(j >= 2)
                def _():
                    o_copy(j - 2, b).wait()

                base = 128 * j

                def i_step(t, _, b=b):
                    i0 = 4 * t
                    prem_rep = plsc.load_gather(prem_all, [base + i0 + dd])
                    rot_rep = plsc.load_gather(rot_all, [base + i0 + dd])
                    row_vec = i0 + dd
                    col_vec = jnp.broadcast_to(i0, (16,)).astype(jnp.int32) + dd
                    for d0 in range(0, _D, 4):
                        skew = prem_rep + ((d0 + rr + rot_rep) & 31)
                        vals = plsc.load_gather(gaths[b], [row_vec, skew])
                        plsc.store_scatter(osts[b], [d0 + rr, col_vec], vals)
                    return _

                lax.fori_loop(0, 32, i_step, None)
                o_copy(j, b).start()

                @pl.when(j + 2 < _P2_WIN)
                def _():
                    g_copy(j + 2, b).start()
            return _

        lax.fori_loop(0, _P2_WIN // 2, p2_step, None)
        o_copy(_P2_WIN - 2, 0).wait()
        o_copy(_P2_WIN - 1, 1).wait()

    return body(tableT, idx)


def kernel(variable_embeddings, candidate_indices):
    idx = candidate_indices.astype(jnp.int32)
    outT, _ = _fused(variable_embeddings.T, idx)
    return outT.T
